# baseline (speedup 1.0000x reference)
import jax
import jax.numpy as jnp
from jax import lax
from jax.experimental import pallas as pl
from jax.experimental.pallas import tpu as pltpu
from jax.experimental.pallas import tpu_sc as plsc

INPUT_DIM = 1000000
PROJ_DIM = 32
BATCH = 16384

_info = plsc.get_sparse_core_info()
_NC, _NS, _L = _info.num_cores, _info.num_subcores, _info.num_lanes
_NW = _NC * _NS
_BPW = BATCH // _NW

_CHUNK = 32768
_NCHUNK = -(-INPUT_DIM // _CHUNK)
_QUART = _CHUNK // 4
_QROWS = _NCHUNK * _QUART
_PROWS = 4 * _QROWS
_LOG_CHUNK = 15
_LOG_QUART = 13

_TCC = 20
_SCC0, _SCC1 = 20, 30
_LO_END = _TCC * _CHUNK
_HI_END = _SCC1 * _CHUNK
_HROWS = _HI_END - _LO_END
_HGROUPS = _HROWS // 4
_WPW = (_SCC1 - _SCC0) * 64 // _NW
_CPC = _CHUNK // 128


def _tc_relayout_body(wwt_ref, wct_ref, pw_ref, pc_ref):
    for src, dst in ((wwt_ref, pw_ref), (wct_ref, pc_ref)):
        x = src[...]
        u = jnp.concatenate(
            [x[:, k * _QUART:(k + 1) * _QUART] for k in range(4)], axis=0)
        dst[...] = jnp.transpose(u)


def _tc_relayout(wwt, wct):
    spec_in = pl.BlockSpec((PROJ_DIM, _CHUNK),
                           lambda c: (0, jnp.where(c < _TCC, c, _NCHUNK - 1)))
    spec_out = pl.BlockSpec((_QUART, 4 * PROJ_DIM),
                            lambda c: (jnp.where(c < _TCC, c, _NCHUNK - 1), 0))
    oshape = jax.ShapeDtypeStruct((_QROWS, 4 * PROJ_DIM), jnp.float32)
    return pl.pallas_call(
        _tc_relayout_body,
        grid=(_TCC + 1,),
        in_specs=[spec_in, spec_in],
        out_specs=[spec_out, spec_out],
        out_shape=[oshape, oshape],
    )(wwt, wct)


def _sc_relayout_body(wwt_hbm, wct_hbm, hw_hbm, hc_hbm, bufin, buf2, semi, semo):
    wid = lax.axis_index("s") * _NC + lax.axis_index("c")
    w0 = wid * _WPW
    lane_iota = lax.iota(jnp.int32, _L)

    for src, dst in ((wwt_hbm, hw_hbm), (wct_hbm, hc_hbm)):
        out_cps = [None, None]
        for i in range(_WPW + 1):
            if i < _WPW:
                wi = w0 + i
                cc = _SCC0 + wi // 64
                m = wi % 64
                p = i % 2
                for k in range(4):
                    col = cc * _CPC + 64 * k + m
                    cb = pl.multiple_of(col * 128, 128)
                    for a in range(4):
                        pltpu.async_copy(
                            src.at[pl.ds(8 * a, 8), pl.ds(cb, 128)],
                            bufin.at[p, pl.ds(32 * k + 8 * a, 8), :], semi)
            if i > 0:
                wi = w0 + i - 1
                cc = _SCC0 + wi // 64
                m = wi % 64
                p = (i - 1) % 2
                pltpu.make_async_copy(
                    hw_hbm.at[pl.ds(0, 128), :], bufin.at[p], semi).wait()
                if out_cps[p] is not None:
                    out_cps[p].wait()

                def tr_body(l, _):
                    lv = jnp.full((_L,), l, jnp.int32)
                    for j0 in range(0, 128, _L):
                        g = plsc.load_gather(
                            bufin, [jnp.full((_L,), p, jnp.int32),
                                    lane_iota + j0, lv])
                        buf2[p, l, pl.ds(j0, _L)] = g
                    return 0

                lax.fori_loop(0, 128, tr_body, 0)
                gl = pl.multiple_of(
                    (cc - _SCC0) * (_CHUNK // 4) + m * 128, 128)
                out_cps[p] = pltpu.async_copy(
                    buf2.at[p], dst.at[pl.ds(gl, 128), :], semo)
        for cp in out_cps:
            if cp is not None:
                cp.wait()


def _sc_relayout(wwt, wct):
    mesh = plsc.VectorSubcoreMesh(core_axis_name="c", subcore_axis_name="s")
    oshape = jax.ShapeDtypeStruct((_HGROUPS, 4 * PROJ_DIM), jnp.float32)
    f = pl.kernel(
        _sc_relayout_body,
        mesh=mesh,
        compiler_params=pltpu.CompilerParams(
            use_tc_tiling_on_sc=True, needs_layout_passes=False),
        out_type=[oshape, oshape],
        scratch_types=[
            pltpu.VMEM((2, 128, 128), jnp.float32),
            pltpu.VMEM((2, 128, 128), jnp.float32),
            pltpu.SemaphoreType.DMA,
            pltpu.SemaphoreType.DMA,
        ],
    )
    return f(wwt, wct)


def _packed_idx(r):
    return (((r >> _LOG_CHUNK) << _LOG_CHUNK)
            + ((r & (_QUART - 1)) << 2) + ((r >> _LOG_QUART) & 3))


def _sc_body(x2_hbm, pw_hbm, pc_hbm, hw_hbm, hc_hbm, out_hbm,
             x_v, il0_v, il1_v, ih0_v, ih1_v, m0_v, m1_v,
             rwl, rwh, rcl, rch, out_v, sem0, sem1):
    wid = lax.axis_index("s") * _NC + lax.axis_index("c")
    base = wid * _BPW

    pltpu.sync_copy(x2_hbm.at[pl.ds(wid * 4, 4), :], x_v)

    for k in range(_BPW // _L):
        t, g = divmod(k * _L, 128)
        sl = pl.ds(k * _L, _L)
        for xoff, il_v, ih_v, m_v in ((g, il0_v, ih0_v, m0_v),
                                      (128 + g, il1_v, ih1_v, m1_v)):
            r = x_v[t, pl.ds(xoff, _L)]
            pp = _packed_idx(r)
            lo = (pp < _LO_END) | (pp >= _HI_END)
            il_v[sl] = jnp.where(lo, pp, 0)
            ih_v[sl] = jnp.where(lo, 0, pp - _LO_END)
            m_v[sl] = jnp.where(lo, 1, 0)

    cps = [pltpu.async_copy(pw_hbm.at[il0_v], rwl, sem0),
           pltpu.async_copy(hw_hbm.at[ih0_v], rwh, sem0),
           pltpu.async_copy(pc_hbm.at[il1_v], rcl, sem1),
           pltpu.async_copy(hc_hbm.at[ih1_v], rch, sem1)]
    for cp in cps:
        cp.wait()

    lane_iota = lax.iota(jnp.int32, _L)

    def block_body(b, _):
        row_idx = lane_iota + b * _L
        sl = pl.ds(b * _L, _L)
        lo0 = m0_v[sl] > 0
        lo1 = m1_v[sl] > 0
        acc = jnp.zeros((_L,), jnp.float32)
        for j in range(PROJ_DIM):
            col = jnp.full((_L,), j, jnp.int32)
            gw = jnp.where(lo0,
                           plsc.load_gather(rwl, [row_idx, col]),
                           plsc.load_gather(rwh, [row_idx, col]))
            gc = jnp.where(lo1,
                           plsc.load_gather(rcl, [row_idx, col]),
                           plsc.load_gather(rch, [row_idx, col]))
            acc = acc + gw * gc
        out_v[sl] = 1.0 / (1.0 + jnp.exp(-acc))
        return 0

    lax.fori_loop(0, _BPW // _L, block_body, 0)

    pltpu.sync_copy(out_v, out_hbm.at[pl.ds(base, _BPW)])


def _sc_gather(x2, pw, pc, hw, hc):
    mesh = plsc.VectorSubcoreMesh(core_axis_name="c", subcore_axis_name="s")
    f = pl.kernel(
        _sc_body,
        mesh=mesh,
        compiler_params=pltpu.CompilerParams(
            use_tc_tiling_on_sc=False, needs_layout_passes=False),
        out_type=jax.ShapeDtypeStruct((BATCH,), jnp.float32),
        scratch_types=[
            pltpu.VMEM((4, 2 * 128), jnp.int32),
            pltpu.VMEM((_BPW,), jnp.int32),
            pltpu.VMEM((_BPW,), jnp.int32),
            pltpu.VMEM((_BPW,), jnp.int32),
            pltpu.VMEM((_BPW,), jnp.int32),
            pltpu.VMEM((_BPW,), jnp.int32),
            pltpu.VMEM((_BPW,), jnp.int32),
            pltpu.VMEM((_BPW, PROJ_DIM), jnp.float32),
            pltpu.VMEM((_BPW, PROJ_DIM), jnp.float32),
            pltpu.VMEM((_BPW, PROJ_DIM), jnp.float32),
            pltpu.VMEM((_BPW, PROJ_DIM), jnp.float32),
            pltpu.VMEM((_BPW,), jnp.float32),
            pltpu.SemaphoreType.DMA,
            pltpu.SemaphoreType.DMA,
        ],
    )
    return f(x2, pw, pc, hw, hc)


@jax.jit
def _run(X, W_w, W_c):
    xt = jnp.reshape(jnp.transpose(X.astype(jnp.int32)), (2, 128, 128))
    x2 = jnp.reshape(jnp.transpose(xt, (1, 0, 2)), (128, 2 * 128))
    wwt = jnp.transpose(W_w)
    wct = jnp.transpose(W_c)
    hw, hc = _sc_relayout(wwt, wct)
    pw, pc = _tc_relayout(wwt, wct)
    out = _sc_gather(x2,
                     jnp.reshape(pw, (_PROWS, PROJ_DIM)),
                     jnp.reshape(pc, (_PROWS, PROJ_DIM)),
                     jnp.reshape(hw, (_HROWS, PROJ_DIM)),
                     jnp.reshape(hc, (_HROWS, PROJ_DIM)))
    return jnp.reshape(out, (BATCH, 1))


def kernel(X, W_w, W_c):
    return _run(X, W_w, W_c)


# speedup vs baseline: 3.6016x; 3.6016x over previous
import jax
import jax.numpy as jnp
from jax import lax
from jax.experimental import pallas as pl
from jax.experimental.pallas import tpu as pltpu
from jax.experimental.pallas import tpu_sc as plsc

INPUT_DIM = 1000000
PROJ_DIM = 32
BATCH = 16384

_info = plsc.get_sparse_core_info()
_NC, _NS, _L = _info.num_cores, _info.num_subcores, _info.num_lanes
_NW = _NC * _NS
_BPW = BATCH // _NW

_CHUNK = 32768
_NCHUNK = -(-INPUT_DIM // _CHUNK)
_QUART = _CHUNK // 4
_QROWS = _NCHUNK * _QUART
_PROWS = 4 * _QROWS


def _relayout_body(wwt_ref, wct_ref, pw_ref, pc_ref):
    for src, dst in ((wwt_ref, pw_ref), (wct_ref, pc_ref)):
        x = src[...]
        u = jnp.concatenate(
            [x[:, k * _QUART:(k + 1) * _QUART] for k in range(4)], axis=0)
        dst[...] = jnp.transpose(u)


def _relayout(wwt, wct):
    spec_in = pl.BlockSpec((PROJ_DIM, _CHUNK), lambda c: (0, c))
    spec_out = pl.BlockSpec((_QUART, 4 * PROJ_DIM), lambda c: (c, 0))
    oshape = jax.ShapeDtypeStruct((_QROWS, 4 * PROJ_DIM), jnp.float32)
    return pl.pallas_call(
        _relayout_body,
        grid=(_NCHUNK,),
        in_specs=[spec_in, spec_in],
        out_specs=[spec_out, spec_out],
        out_shape=[oshape, oshape],
    )(wwt, wct)


_LOG_CHUNK = _CHUNK.bit_length() - 1
_LOG_QUART = _QUART.bit_length() - 1


def _packed_idx(r):
    return (((r >> _LOG_CHUNK) << _LOG_CHUNK)
            + ((r & (_QUART - 1)) << 2) + ((r >> _LOG_QUART) & 3))


def _sc_body(x2_hbm, pw_hbm, pc_hbm, out_hbm,
             x_v, idx0_v, idx1_v, rows_w, rows_c, out_v, sem0, sem1):
    wid = lax.axis_index("s") * _NC + lax.axis_index("c")
    base = wid * _BPW

    pltpu.sync_copy(x2_hbm.at[pl.ds(wid * 4, 4), :], x_v)

    for k in range(_BPW // _L):
        t, g = divmod(k * _L, 128)
        r0 = x_v[t, pl.ds(g, _L)]
        r1 = x_v[t, pl.ds(128 + g, _L)]
        idx0_v[pl.ds(k * _L, _L)] = _packed_idx(r0)
        idx1_v[pl.ds(k * _L, _L)] = _packed_idx(r1)

    cp0 = pltpu.async_copy(pw_hbm.at[idx0_v], rows_w, sem0)
    cp1 = pltpu.async_copy(pc_hbm.at[idx1_v], rows_c, sem1)
    cp0.wait()
    cp1.wait()

    lane_iota = lax.iota(jnp.int32, _L)

    def block_body(b, _):
        row_idx = lane_iota + b * _L
        acc = jnp.zeros((_L,), jnp.float32)
        for j in range(PROJ_DIM):
            col = jnp.full((_L,), j, jnp.int32)
            gw = plsc.load_gather(rows_w, [row_idx, col])
            gc = plsc.load_gather(rows_c, [row_idx, col])
            acc = acc + gw * gc
        out_v[pl.ds(b * _L, _L)] = 1.0 / (1.0 + jnp.exp(-acc))
        return 0

    lax.fori_loop(0, _BPW // _L, block_body, 0)

    pltpu.sync_copy(out_v, out_hbm.at[pl.ds(base, _BPW)])


def _sc_gather(x2, pw, pc):
    mesh = plsc.VectorSubcoreMesh(core_axis_name="c", subcore_axis_name="s")
    f = pl.kernel(
        _sc_body,
        mesh=mesh,
        compiler_params=pltpu.CompilerParams(
            use_tc_tiling_on_sc=False, needs_layout_passes=False),
        out_type=jax.ShapeDtypeStruct((BATCH,), jnp.float32),
        scratch_types=[
            pltpu.VMEM((4, 2 * 128), jnp.int32),
            pltpu.VMEM((_BPW,), jnp.int32),
            pltpu.VMEM((_BPW,), jnp.int32),
            pltpu.VMEM((_BPW, PROJ_DIM), jnp.float32),
            pltpu.VMEM((_BPW, PROJ_DIM), jnp.float32),
            pltpu.VMEM((_BPW,), jnp.float32),
            pltpu.SemaphoreType.DMA,
            pltpu.SemaphoreType.DMA,
        ],
    )
    return f(x2, pw, pc)


@jax.jit
def _run(X, W_w, W_c):
    xt = jnp.reshape(jnp.transpose(X.astype(jnp.int32)), (2, 128, 128))
    x2 = jnp.reshape(jnp.transpose(xt, (1, 0, 2)), (128, 2 * 128))
    pw, pc = _relayout(jnp.transpose(W_w), jnp.transpose(W_c))
    out = _sc_gather(x2,
                     jnp.reshape(pw, (_PROWS, PROJ_DIM)),
                     jnp.reshape(pc, (_PROWS, PROJ_DIM)))
    return jnp.reshape(out, (BATCH, 1))


def kernel(X, W_w, W_c):
    return _run(X, W_w, W_c)
